# baseline (speedup 1.0000x reference)
import functools

import jax
import jax.numpy as jnp
from jax import lax
from jax.experimental import pallas as pl
from jax.experimental.pallas import tpu as pltpu
from jax.experimental.pallas import tpu_sc as plsc

M = 320000
D = 128
T = 1280000

NC = 2
NS = 16
L = 16

R_USE = 10000
NCHUNK = M // R_USE
CHUNK_PER_SC = NCHUNK // NC
ACC_ROWS = 10240
ZROWS = 40
ZREP = 16
GARBAGE_ROW = R_USE

TPW = T // NS
BLK_T = 1600
assert BLK_T % (2 * L) == 0 and (T // NS) % BLK_T == 0
NBLK = TPW // BLK_T
C = 128
FLUSH_AT = C - L

MLP_BLK = 2000


def _mlp_body(x_ref, w1a, b1a, w2a, b2a, w1b, b1b, w2b, b2b, o1_ref, o2_ref):
    x = x_ref[...]
    h = jnp.maximum(jnp.dot(x, w1a[...], preferred_element_type=jnp.float32)
                    + b1a[...], 0.0)
    o1_ref[...] = jnp.dot(h, w2a[...], preferred_element_type=jnp.float32) + b2a[...]
    h = jnp.maximum(jnp.dot(x, w1b[...], preferred_element_type=jnp.float32)
                    + b1b[...], 0.0)
    o2_ref[...] = jnp.dot(h, w2b[...], preferred_element_type=jnp.float32) + b2b[...]


def _mlps(X, W1a, b1a, W2a, b2a, W1b, b1b, W2b, b2b):
    n = X.shape[0]
    w_spec = pl.BlockSpec((D, D), lambda i: (0, 0))
    b_spec = pl.BlockSpec((1, D), lambda i: (0, 0))
    x_spec = pl.BlockSpec((MLP_BLK, D), lambda i: (i, 0))
    return pl.pallas_call(
        _mlp_body,
        grid=(n // MLP_BLK,),
        in_specs=[x_spec, w_spec, b_spec, w_spec, b_spec,
                  w_spec, b_spec, w_spec, b_spec],
        out_specs=[x_spec, x_spec],
        out_shape=[jax.ShapeDtypeStruct((n, D), jnp.float32),
                   jax.ShapeDtypeStruct((n, D), jnp.float32)],
    )(X, W1a, b1a.reshape(1, D), W2a, b2a.reshape(1, D),
      W1b, b1b.reshape(1, D), W2b, b2b.reshape(1, D))


def _msgpass_body(x1_hbm, x2_hbm, tar_hbm, s1_hbm, s2_hbm, zeros_hbm, out_hbm,
                  tarv, s1v, s2v, mtar, ms1, ms2, rows1, rows2,
                  acc, sem1, sem2, sem3, gsem1, gsem2):
    core = lax.axis_index("c")
    sub = lax.axis_index("s")
    zvec_i = jnp.zeros((L,), jnp.int32)
    gvec_i = jnp.full((L,), GARBAGE_ROW, jnp.int32)

    rows_per_sub_acc = ACC_ROWS // NS

    def zero_own_region():
        pltpu.sync_copy(zeros_hbm,
                        acc.at[pl.ds(sub * rows_per_sub_acc, rows_per_sub_acc)])

    def refill_buffers(q):
        for j in range(C // L):
            sl = pl.ds(j * L, L)
            mtar[q, sl] = gvec_i
            ms1[q, sl] = zvec_i
            ms2[q, sl] = zvec_i

    def issue_gathers(q):
        pltpu.async_copy(x1_hbm.at[ms1.at[q]], rows1, gsem1)
        pltpu.async_copy(x2_hbm.at[ms2.at[q]], rows2, gsem2)

    def drain_process(q):
        pltpu.make_async_copy(x1_hbm.at[ms1.at[q]], rows1, gsem1).wait()
        pltpu.make_async_copy(x2_hbm.at[ms2.at[q]], rows2, gsem2).wait()

        def mul_body(r4, c):
            for u in range(4):
                r = r4 * 4 + u
                for v in range(D // L):
                    sl = pl.ds(v * L, L)
                    rows1[r, sl] = rows1[r, sl] * rows2[r, sl]
            return c
        lax.fori_loop(0, C // 4, mul_body, 0)
        pltpu.sync_copy(rows1, acc.at[mtar.at[q]], add=True)
        refill_buffers(q)

    zero_own_region()
    refill_buffers(0)
    refill_buffers(1)
    plsc.subcore_barrier()

    iota = lax.iota(jnp.int32, L)

    def pass_body(p, _):
        chunk = core * CHUNK_PER_SC + p
        lo = chunk * R_USE

        off0 = sub * TPW
        pltpu.async_copy(tar_hbm.at[pl.ds(off0, BLK_T)], tarv.at[0], sem1)
        pltpu.async_copy(s1_hbm.at[pl.ds(off0, BLK_T)], s1v.at[0], sem2)
        pltpu.async_copy(s2_hbm.at[pl.ds(off0, BLK_T)], s2v.at[0], sem3)

        def blk_body(b, carry):
            qb = lax.rem(b, 2)
            pltpu.make_async_copy(tar_hbm.at[pl.ds(0, BLK_T)],
                                  tarv.at[qb], sem1).wait()
            pltpu.make_async_copy(s1_hbm.at[pl.ds(0, BLK_T)],
                                  s1v.at[qb], sem2).wait()
            pltpu.make_async_copy(s2_hbm.at[pl.ds(0, BLK_T)],
                                  s2v.at[qb], sem3).wait()

            @pl.when(b + 1 < NBLK)
            def _():
                off2 = sub * TPW + (b + 1) * BLK_T
                pltpu.async_copy(tar_hbm.at[pl.ds(off2, BLK_T)],
                                 tarv.at[1 - qb], sem1)
                pltpu.async_copy(s1_hbm.at[pl.ds(off2, BLK_T)],
                                 s1v.at[1 - qb], sem2)
                pltpu.async_copy(s2_hbm.at[pl.ds(off2, BLK_T)],
                                 s2v.at[1 - qb], sem3)

            def scan_one(ii, state):
                wp, par, pend = state
                sl = pl.ds(ii * L, L)
                rel = tarv[qb, sl] - lo
                m = plsc.bitcast(rel, jnp.uint32) < jnp.uint32(R_USE)
                cnt = plsc.all_reduce_population_count(m)[0]
                has = cnt > 0
                need_flush = has & (wp > FLUSH_AT)

                @pl.when(need_flush)
                def _():
                    @pl.when(pend == 1)
                    def _():
                        drain_process(1 - par)
                    issue_gathers(par)

                par = jnp.where(need_flush, 1 - par, par)
                pend = jnp.where(need_flush, 1, pend)
                wp = jnp.where(need_flush, 0, wp)

                @pl.when(has)
                def _():
                    _, relC, mo = plsc.sort_key_val(iota, rel, mask=m)
                    _, s1C, _ = plsc.sort_key_val(iota, s1v[qb, sl], mask=m)
                    _, s2C, _ = plsc.sort_key_val(iota, s2v[qb, sl], mask=m)
                    mtar[par, pl.ds(wp, L)] = jnp.where(mo, relC, GARBAGE_ROW)
                    ms1[par, pl.ds(wp, L)] = jnp.where(mo, s1C, 0)
                    ms2[par, pl.ds(wp, L)] = jnp.where(mo, s2C, 0)

                return (wp + cnt, par, pend)

            def scan_iter(i, state):
                for h in range(2):
                    state = scan_one(i * 2 + h, state)
                return state

            return lax.fori_loop(0, BLK_T // (2 * L), scan_iter, carry)

        wp, par, pend = lax.fori_loop(0, NBLK, blk_body, (0, 0, 0))

        @pl.when(pend == 1)
        def _():
            drain_process(1 - par)

        @pl.when(wp > 0)
        def _():
            issue_gathers(par)
            drain_process(par)

        plsc.subcore_barrier()
        rows_per_sub = R_USE // NS
        pltpu.sync_copy(acc.at[pl.ds(sub * rows_per_sub, rows_per_sub)],
                        out_hbm.at[pl.ds(lo + sub * rows_per_sub, rows_per_sub)])
        plsc.subcore_barrier()
        zero_own_region()
        plsc.subcore_barrier()
        return 0

    lax.fori_loop(0, CHUNK_PER_SC, pass_body, 0)


def _msgpass(X1, X2, tar_ind, src1_ind, src2_ind):
    mesh = plsc.VectorSubcoreMesh(core_axis_name="c", subcore_axis_name="s")
    f = pl.kernel(
        _msgpass_body,
        out_type=jax.ShapeDtypeStruct((M, D), jnp.float32),
        mesh=mesh,
        compiler_params=pltpu.CompilerParams(needs_layout_passes=False,
                                             use_tc_tiling_on_sc=False),
        scratch_types=[
            pltpu.VMEM((2, BLK_T), jnp.int32),
            pltpu.VMEM((2, BLK_T), jnp.int32),
            pltpu.VMEM((2, BLK_T), jnp.int32),
            pltpu.VMEM((2, C), jnp.int32),
            pltpu.VMEM((2, C), jnp.int32),
            pltpu.VMEM((2, C), jnp.int32),
            pltpu.VMEM((C, D), jnp.float32),
            pltpu.VMEM((C, D), jnp.float32),
            pltpu.VMEM_SHARED((ACC_ROWS, D), jnp.float32),
            pltpu.SemaphoreType.DMA,
            pltpu.SemaphoreType.DMA,
            pltpu.SemaphoreType.DMA,
            pltpu.SemaphoreType.DMA,
            pltpu.SemaphoreType.DMA,
        ],
    )
    zeros = jnp.zeros((ACC_ROWS // NS, D), jnp.float32)
    return f(X1, X2, tar_ind, src1_ind, src2_ind, zeros)


def kernel(X, tar_ind, src1_ind, src2_ind,
           W1a, b1a, W2a, b2a, W1b, b1b, W2b, b2b):
    assert X.shape == (M, D) and tar_ind.shape == (T,)
    X1, X2 = _mlps(X, W1a, b1a, W2a, b2a, W1b, b1b, W2b, b2b)
    return _msgpass(X1, X2,
                    tar_ind.astype(jnp.int32),
                    src1_ind.astype(jnp.int32),
                    src2_ind.astype(jnp.int32))


# speedup vs baseline: 1.0711x; 1.0711x over previous
import functools

import jax
import jax.numpy as jnp
from jax import lax
from jax.experimental import pallas as pl
from jax.experimental.pallas import tpu as pltpu
from jax.experimental.pallas import tpu_sc as plsc

M = 320000
D = 128
T = 1280000

NC = 2
NS = 16
L = 16

R_USE = 8000
NCHUNK = M // R_USE
CHUNK_PER_SC = NCHUNK // NC
ACC_ROWS = 8192
GARBAGE_ROW = R_USE

TPW = T // NS
BLK_T = 1600
assert BLK_T % (2 * L) == 0 and (T // NS) % BLK_T == 0
NBLK = TPW // BLK_T
C = 192
FLUSH_AT = C - L

MLP_BLK = 2000


def _mlp_body(x_ref, w1a, b1a, w2a, b2a, w1b, b1b, w2b, b2b, o1_ref, o2_ref):
    x = x_ref[...]
    h = jnp.maximum(jnp.dot(x, w1a[...], preferred_element_type=jnp.float32)
                    + b1a[...], 0.0)
    o1_ref[...] = jnp.dot(h, w2a[...], preferred_element_type=jnp.float32) + b2a[...]
    h = jnp.maximum(jnp.dot(x, w1b[...], preferred_element_type=jnp.float32)
                    + b1b[...], 0.0)
    o2_ref[...] = jnp.dot(h, w2b[...], preferred_element_type=jnp.float32) + b2b[...]


def _mlps(X, W1a, b1a, W2a, b2a, W1b, b1b, W2b, b2b):
    n = X.shape[0]
    w_spec = pl.BlockSpec((D, D), lambda i: (0, 0))
    b_spec = pl.BlockSpec((1, D), lambda i: (0, 0))
    x_spec = pl.BlockSpec((MLP_BLK, D), lambda i: (i, 0))
    return pl.pallas_call(
        _mlp_body,
        grid=(n // MLP_BLK,),
        in_specs=[x_spec, w_spec, b_spec, w_spec, b_spec,
                  w_spec, b_spec, w_spec, b_spec],
        out_specs=[x_spec, x_spec],
        out_shape=[jax.ShapeDtypeStruct((n, D), jnp.float32),
                   jax.ShapeDtypeStruct((n, D), jnp.float32)],
    )(X, W1a, b1a.reshape(1, D), W2a, b2a.reshape(1, D),
      W1b, b1b.reshape(1, D), W2b, b2b.reshape(1, D))


def _msgpass_body(x1_hbm, x2_hbm, tar_hbm, s1_hbm, s2_hbm, zeros_hbm, out_hbm,
                  tarv, s1v, s2v, mtar, ms1, ms2, rows1, rows2,
                  acc, sem1, sem2, sem3, gsem1, gsem2):
    core = lax.axis_index("c")
    sub = lax.axis_index("s")
    zvec_i = jnp.zeros((L,), jnp.int32)
    gvec_i = jnp.full((L,), GARBAGE_ROW, jnp.int32)

    rows_per_sub_acc = ACC_ROWS // NS

    def zero_own_region():
        pltpu.sync_copy(zeros_hbm,
                        acc.at[pl.ds(sub * rows_per_sub_acc, rows_per_sub_acc)])

    def refill_buffers(q):
        for j in range(C // L):
            sl = pl.ds(j * L, L)
            mtar[q, sl] = gvec_i
            ms1[q, sl] = zvec_i
            ms2[q, sl] = zvec_i

    def issue_gathers(q):
        pltpu.async_copy(x1_hbm.at[ms1.at[q]], rows1, gsem1)
        pltpu.async_copy(x2_hbm.at[ms2.at[q]], rows2, gsem2)

    def drain_process(q):
        pltpu.make_async_copy(x1_hbm.at[ms1.at[q]], rows1, gsem1).wait()
        pltpu.make_async_copy(x2_hbm.at[ms2.at[q]], rows2, gsem2).wait()

        def mul_body(r4, c):
            for u in range(4):
                r = r4 * 4 + u
                for v in range(D // L):
                    sl = pl.ds(v * L, L)
                    rows1[r, sl] = rows1[r, sl] * rows2[r, sl]
            return c
        lax.fori_loop(0, C // 4, mul_body, 0)
        pltpu.sync_copy(rows1, acc.at[mtar.at[q]], add=True)
        refill_buffers(q)

    zero_own_region()
    refill_buffers(0)
    refill_buffers(1)
    plsc.subcore_barrier()

    iota = lax.iota(jnp.int32, L)

    def pass_body(p, _):
        chunk = core * CHUNK_PER_SC + p
        lo = chunk * R_USE

        off0 = sub * TPW
        pltpu.async_copy(tar_hbm.at[pl.ds(off0, BLK_T)], tarv.at[0], sem1)
        pltpu.async_copy(s1_hbm.at[pl.ds(off0, BLK_T)], s1v.at[0], sem2)
        pltpu.async_copy(s2_hbm.at[pl.ds(off0, BLK_T)], s2v.at[0], sem3)

        def blk_body(b, carry):
            qb = lax.rem(b, 2)
            pltpu.make_async_copy(tar_hbm.at[pl.ds(0, BLK_T)],
                                  tarv.at[qb], sem1).wait()
            pltpu.make_async_copy(s1_hbm.at[pl.ds(0, BLK_T)],
                                  s1v.at[qb], sem2).wait()
            pltpu.make_async_copy(s2_hbm.at[pl.ds(0, BLK_T)],
                                  s2v.at[qb], sem3).wait()

            @pl.when(b + 1 < NBLK)
            def _():
                off2 = sub * TPW + (b + 1) * BLK_T
                pltpu.async_copy(tar_hbm.at[pl.ds(off2, BLK_T)],
                                 tarv.at[1 - qb], sem1)
                pltpu.async_copy(s1_hbm.at[pl.ds(off2, BLK_T)],
                                 s1v.at[1 - qb], sem2)
                pltpu.async_copy(s2_hbm.at[pl.ds(off2, BLK_T)],
                                 s2v.at[1 - qb], sem3)

            def scan_one(ii, state):
                wp, par, pend = state
                sl = pl.ds(ii * L, L)
                rel = tarv[qb, sl] - lo
                m = plsc.bitcast(rel, jnp.uint32) < jnp.uint32(R_USE)
                cnt = plsc.all_reduce_population_count(m)[0]
                has = cnt > 0
                need_flush = has & (wp > FLUSH_AT)

                @pl.when(need_flush)
                def _():
                    @pl.when(pend == 1)
                    def _():
                        drain_process(1 - par)
                    issue_gathers(par)

                par = jnp.where(need_flush, 1 - par, par)
                pend = jnp.where(need_flush, 1, pend)
                wp = jnp.where(need_flush, 0, wp)

                @pl.when(has)
                def _():
                    _, relC, mo = plsc.sort_key_val(iota, rel, mask=m)
                    _, s1C, _ = plsc.sort_key_val(iota, s1v[qb, sl], mask=m)
                    _, s2C, _ = plsc.sort_key_val(iota, s2v[qb, sl], mask=m)
                    mtar[par, pl.ds(wp, L)] = jnp.where(mo, relC, GARBAGE_ROW)
                    ms1[par, pl.ds(wp, L)] = jnp.where(mo, s1C, 0)
                    ms2[par, pl.ds(wp, L)] = jnp.where(mo, s2C, 0)

                return (wp + cnt, par, pend)

            def scan_iter(i, state):
                for h in range(2):
                    state = scan_one(i * 2 + h, state)
                return state

            return lax.fori_loop(0, BLK_T // (2 * L), scan_iter, carry)

        wp, par, pend = lax.fori_loop(0, NBLK, blk_body, (0, 0, 0))

        @pl.when(pend == 1)
        def _():
            drain_process(1 - par)

        @pl.when(wp > 0)
        def _():
            issue_gathers(par)
            drain_process(par)

        plsc.subcore_barrier()
        rows_per_sub = R_USE // NS
        pltpu.sync_copy(acc.at[pl.ds(sub * rows_per_sub, rows_per_sub)],
                        out_hbm.at[pl.ds(lo + sub * rows_per_sub, rows_per_sub)])
        plsc.subcore_barrier()
        zero_own_region()
        plsc.subcore_barrier()
        return 0

    lax.fori_loop(0, CHUNK_PER_SC, pass_body, 0)


def _msgpass(X1, X2, tar_ind, src1_ind, src2_ind):
    mesh = plsc.VectorSubcoreMesh(core_axis_name="c", subcore_axis_name="s")
    f = pl.kernel(
        _msgpass_body,
        out_type=jax.ShapeDtypeStruct((M, D), jnp.float32),
        mesh=mesh,
        compiler_params=pltpu.CompilerParams(needs_layout_passes=False,
                                             use_tc_tiling_on_sc=False),
        scratch_types=[
            pltpu.VMEM((2, BLK_T), jnp.int32),
            pltpu.VMEM((2, BLK_T), jnp.int32),
            pltpu.VMEM((2, BLK_T), jnp.int32),
            pltpu.VMEM((2, C), jnp.int32),
            pltpu.VMEM((2, C), jnp.int32),
            pltpu.VMEM((2, C), jnp.int32),
            pltpu.VMEM((C, D), jnp.float32),
            pltpu.VMEM((C, D), jnp.float32),
            pltpu.VMEM_SHARED((ACC_ROWS, D), jnp.float32),
            pltpu.SemaphoreType.DMA,
            pltpu.SemaphoreType.DMA,
            pltpu.SemaphoreType.DMA,
            pltpu.SemaphoreType.DMA,
            pltpu.SemaphoreType.DMA,
        ],
    )
    zeros = jnp.zeros((ACC_ROWS // NS, D), jnp.float32)
    return f(X1, X2, tar_ind, src1_ind, src2_ind, zeros)


def kernel(X, tar_ind, src1_ind, src2_ind,
           W1a, b1a, W2a, b2a, W1b, b1b, W2b, b2b):
    assert X.shape == (M, D) and tar_ind.shape == (T,)
    X1, X2 = _mlps(X, W1a, b1a, W2a, b2a, W1b, b1b, W2b, b2b)
    return _msgpass(X1, X2,
                    tar_ind.astype(jnp.int32),
                    src1_ind.astype(jnp.int32),
                    src2_ind.astype(jnp.int32))
